# initial kernel scaffold (unmeasured)
import jax
import jax.numpy as jnp
from jax import lax
from jax.experimental import pallas as pl
from jax.experimental.pallas import tpu as pltpu

N_DEV = 4
N_TOK = 2048
D = 1024
H = 1024
E_LOCAL = 8
BLK = N_TOK // N_DEV


def kernel(x, router_W, route_idx, expert_W, shared_W):
    scores = jnp.dot(x, router_W, precision=lax.Precision.HIGHEST)
    probs = jax.nn.softmax(scores, axis=-1)
    gate = jnp.take_along_axis(probs, route_idx, axis=1)

    my = lax.axis_index("i")
    local_e = route_idx - my * E_LOCAL
    onehot = local_e == jnp.arange(E_LOCAL)[None, :]
    coefs = jnp.where(onehot, gate, 0.0).astype(jnp.float32)

    def body(x_ref, coefs_ref, expert_W_ref, shared_W_ref, out_ref,
             wbuf, send_ref, comm_ref, load_sems, send_sems, recv_sems):
        my_pos = lax.axis_index("i")
        left = lax.rem(my_pos + N_DEV - 1, N_DEV)
        right = lax.rem(my_pos + 1, N_DEV)

        loads = []
        for j in range(E_LOCAL):
            cp = pltpu.make_async_copy(
                expert_W_ref.at[j], wbuf.at[j], load_sems.at[j])
            cp.start()
            loads.append(cp)
        waited = [False] * E_LOCAL

        barrier_sem = pltpu.get_barrier_semaphore()
        for nbr in (left, right):
            pl.semaphore_signal(
                barrier_sem, inc=1,
                device_id=(nbr,), device_id_type=pl.DeviceIdType.MESH)
        pl.semaphore_wait(barrier_sem, 2)

        def block_partial(c):
            rows = pl.ds(c * BLK, BLK)
            xblk = x_ref[rows, :]
            cblk = coefs_ref[rows, :]
            acc = jnp.zeros((BLK, H), jnp.float32)
            for j in range(E_LOCAL):
                if not waited[j]:
                    loads[j].wait()
                    waited[j] = True
                xm = xblk * cblk[:, j][:, None]
                acc = acc + jnp.dot(xm, wbuf[j],
                                    preferred_element_type=jnp.float32)
            return acc

        for s in range(N_DEV - 1):
            c = lax.rem(my_pos + N_DEV - 1 - s, N_DEV)
            val = block_partial(c)
            if s > 0:
                val = val + comm_ref[s - 1]
            send_ref[...] = val
            rdma = pltpu.make_async_remote_copy(
                src_ref=send_ref,
                dst_ref=comm_ref.at[s],
                send_sem=send_sems.at[s],
                recv_sem=recv_sems.at[s],
                device_id=(right,),
                device_id_type=pl.DeviceIdType.MESH,
            )
            rdma.start()
            rdma.wait()

        own = block_partial(my_pos)
        own_rows = pl.ds(my_pos * BLK, BLK)
        shared_own = jnp.dot(x_ref[own_rows, :], shared_W_ref[...],
                             preferred_element_type=jnp.float32)
        out_ref[...] = own + comm_ref[N_DEV - 2] + shared_own

    return pl.pallas_call(
        body,
        out_shape=jax.ShapeDtypeStruct((BLK, H), jnp.float32),
        in_specs=[
            pl.BlockSpec(memory_space=pltpu.VMEM),
            pl.BlockSpec(memory_space=pltpu.VMEM),
            pl.BlockSpec(memory_space=pltpu.ANY),
            pl.BlockSpec(memory_space=pltpu.VMEM),
        ],
        out_specs=pl.BlockSpec(memory_space=pltpu.VMEM),
        scratch_shapes=[
            pltpu.VMEM((E_LOCAL, D, H), jnp.float32),
            pltpu.VMEM((BLK, H), jnp.float32),
            pltpu.VMEM((N_DEV - 1, BLK, H), jnp.float32),
            pltpu.SemaphoreType.DMA((E_LOCAL,)),
            pltpu.SemaphoreType.DMA((N_DEV - 1,)),
            pltpu.SemaphoreType.DMA((N_DEV - 1,)),
        ],
        compiler_params=pltpu.CompilerParams(collective_id=0),
    )(x, coefs, expert_W, shared_W)


# baseline (device time: 167350 ns/iter reference)
import jax
import jax.numpy as jnp
from jax import lax
from jax.experimental import pallas as pl
from jax.experimental.pallas import tpu as pltpu

N_DEV = 4
N_TOK = 2048
D = 1024
H = 1024
E_LOCAL = 8
BLK = N_TOK // N_DEV


def kernel(x, router_W, route_idx, expert_W, shared_W):
    scores = jnp.dot(x, router_W, precision=lax.Precision.HIGHEST)
    probs = jax.nn.softmax(scores, axis=-1)
    gate = jnp.take_along_axis(probs, route_idx, axis=1)

    my = lax.axis_index("i")
    local_e = route_idx - my * E_LOCAL
    onehot = local_e == jnp.arange(E_LOCAL)[None, :]
    coefs = jnp.where(onehot, gate, 0.0).astype(jnp.float32)

    def body(x_ref, coefs_ref, expert_W_ref, shared_W_ref, out_ref,
             wbuf, send_ref, comm_ref, load_sems, send_sems, recv_sems):
        my_pos = lax.axis_index("i")
        left = lax.rem(my_pos + N_DEV - 1, N_DEV)
        right = lax.rem(my_pos + 1, N_DEV)

        barrier_sem = pltpu.get_barrier_semaphore()
        for nbr in (left, right):
            pl.semaphore_signal(
                barrier_sem, inc=1,
                device_id=(nbr,), device_id_type=pl.DeviceIdType.MESH)
        pl.semaphore_wait(barrier_sem, 2)

        def wload(j, slot):
            return pltpu.make_async_copy(
                expert_W_ref.at[j], wbuf.at[slot], load_sems.at[slot])

        def block_partial(c):
            rows = pl.ds(c * BLK, BLK)
            xblk = x_ref[rows, :]
            cblk = coefs_ref[rows, :]
            wload(0, 0).start()
            acc = jnp.zeros((BLK, H), jnp.float32)
            for j in range(E_LOCAL):
                if j + 1 < E_LOCAL:
                    wload(j + 1, (j + 1) % 2).start()
                wload(j, j % 2).wait()
                xm = xblk * cblk[:, j][:, None]
                acc = acc + jnp.dot(xm, wbuf[j % 2],
                                    preferred_element_type=jnp.float32)
            return acc

        for s in range(N_DEV - 1):
            c = lax.rem(my_pos + N_DEV - 1 - s, N_DEV)
            val = block_partial(c)
            if s > 0:
                val = val + comm_ref[s - 1]
            send_ref[...] = val
            rdma = pltpu.make_async_remote_copy(
                src_ref=send_ref,
                dst_ref=comm_ref.at[s],
                send_sem=send_sems.at[s],
                recv_sem=recv_sems.at[s],
                device_id=(right,),
                device_id_type=pl.DeviceIdType.MESH,
            )
            rdma.start()
            rdma.wait()

        own = block_partial(my_pos)
        own_rows = pl.ds(my_pos * BLK, BLK)
        shared_own = jnp.dot(x_ref[own_rows, :], shared_W_ref[...],
                             preferred_element_type=jnp.float32)
        out_ref[...] = own + comm_ref[N_DEV - 2] + shared_own

    return pl.pallas_call(
        body,
        out_shape=jax.ShapeDtypeStruct((BLK, H), jnp.float32),
        in_specs=[
            pl.BlockSpec(memory_space=pltpu.VMEM),
            pl.BlockSpec(memory_space=pltpu.VMEM),
            pl.BlockSpec(memory_space=pl.ANY),
            pl.BlockSpec(memory_space=pltpu.VMEM),
        ],
        out_specs=pl.BlockSpec(memory_space=pltpu.VMEM),
        scratch_shapes=[
            pltpu.VMEM((2, D, H), jnp.float32),
            pltpu.VMEM((BLK, H), jnp.float32),
            pltpu.VMEM((N_DEV - 1, BLK, H), jnp.float32),
            pltpu.SemaphoreType.DMA((2,)),
            pltpu.SemaphoreType.DMA((N_DEV - 1,)),
            pltpu.SemaphoreType.DMA((N_DEV - 1,)),
        ],
        compiler_params=pltpu.CompilerParams(
            collective_id=0, vmem_limit_bytes=64 * 1024 * 1024),
    )(x, coefs, expert_W, shared_W)


# device time: 111086 ns/iter; 1.5065x vs baseline; 1.5065x over previous
import jax
import jax.numpy as jnp
from jax import lax
from jax.experimental import pallas as pl
from jax.experimental.pallas import tpu as pltpu

N_DEV = 4
N_TOK = 2048
D = 1024
H = 1024
E_LOCAL = 8
E_TOT = N_DEV * E_LOCAL
BLK = N_TOK // N_DEV


def kernel(x, router_W, route_idx, expert_W, shared_W):
    def body(x_ref, router_W_ref, route_idx_ref, expert_W_ref, shared_W_ref,
             out_ref, wbuf, coef_ref, sbuf, rbuf, load_sems, send_sems,
             recv_sems):
        my_pos = lax.axis_index("i")

        barrier_sem = pltpu.get_barrier_semaphore()
        peers = [lax.rem(my_pos + k, N_DEV) for k in (1, 2, 3)]
        for nbr in peers:
            pl.semaphore_signal(
                barrier_sem, inc=1,
                device_id=(nbr,), device_id_type=pl.DeviceIdType.MESH)

        xf = x_ref[...]
        xh = xf.astype(jnp.bfloat16)
        xl = (xf - xh.astype(jnp.float32)).astype(jnp.bfloat16)
        rw = router_W_ref[...]
        rh = rw.astype(jnp.bfloat16)
        rl = (rw - rh.astype(jnp.float32)).astype(jnp.bfloat16)
        scores = (
            jnp.dot(xh, rh, preferred_element_type=jnp.float32)
            + jnp.dot(xh, rl, preferred_element_type=jnp.float32)
            + jnp.dot(xl, rh, preferred_element_type=jnp.float32)
        )
        m = jnp.max(scores, axis=1, keepdims=True)
        p = jnp.exp(scores - m)
        psum = jnp.sum(p, axis=1, keepdims=True)
        ridx = route_idx_ref[...]
        iota_e = lax.broadcasted_iota(jnp.int32, (N_TOK, E_TOT), 1)
        gate = jnp.sum(jnp.where(iota_e == ridx, p, 0.0), axis=1,
                       keepdims=True) / psum
        iota_l = lax.broadcasted_iota(jnp.int32, (N_TOK, E_LOCAL), 1)
        coef_ref[...] = jnp.where(
            iota_l + my_pos * E_LOCAL == ridx, gate, 0.0)

        pl.semaphore_wait(barrier_sem, N_DEV - 1)

        def wload(j, slot):
            return pltpu.make_async_copy(
                expert_W_ref.at[j], wbuf.at[slot], load_sems.at[slot])

        def block_partial(c):
            rows = pl.ds(c * BLK, BLK)
            xblk = x_ref[rows, :]
            cblk = coef_ref[rows, :]
            wload(0, 0).start()
            acc = jnp.zeros((BLK, H), jnp.float32)
            for j in range(E_LOCAL):
                if j + 1 < E_LOCAL:
                    wload(j + 1, (j + 1) % 2).start()
                wload(j, j % 2).wait()
                xm = xblk * cblk[:, j][:, None]
                acc = acc + jnp.dot(xm, wbuf[j % 2],
                                    preferred_element_type=jnp.float32)
            return acc

        sends = []
        for i, k in enumerate((2, 1, 3)):
            dest = lax.rem(my_pos + k, N_DEV)
            acc = block_partial(dest)
            sbuf[i] = acc.astype(jnp.bfloat16)
            slot = k - 1
            rdma = pltpu.make_async_remote_copy(
                src_ref=sbuf.at[i],
                dst_ref=rbuf.at[slot],
                send_sem=send_sems.at[i],
                recv_sem=recv_sems.at[slot],
                device_id=(dest,),
                device_id_type=pl.DeviceIdType.MESH,
            )
            rdma.start()
            sends.append(rdma)

        own = block_partial(my_pos)
        own_rows = pl.ds(my_pos * BLK, BLK)
        shared_own = jnp.dot(x_ref[own_rows, :], shared_W_ref[...],
                             preferred_element_type=jnp.float32)

        for slot in range(N_DEV - 1):
            sends[slot].wait_recv()
        total = own + shared_own
        for slot in range(N_DEV - 1):
            total = total + rbuf[slot].astype(jnp.float32)
        out_ref[...] = total
        for s in sends:
            s.wait_send()

    return pl.pallas_call(
        body,
        out_shape=jax.ShapeDtypeStruct((BLK, H), jnp.float32),
        in_specs=[
            pl.BlockSpec(memory_space=pltpu.VMEM),
            pl.BlockSpec(memory_space=pltpu.VMEM),
            pl.BlockSpec(memory_space=pltpu.VMEM),
            pl.BlockSpec(memory_space=pl.ANY),
            pl.BlockSpec(memory_space=pltpu.VMEM),
        ],
        out_specs=pl.BlockSpec(memory_space=pltpu.VMEM),
        scratch_shapes=[
            pltpu.VMEM((2, D, H), jnp.float32),
            pltpu.VMEM((N_TOK, E_LOCAL), jnp.float32),
            pltpu.VMEM((N_DEV - 1, BLK, H), jnp.bfloat16),
            pltpu.VMEM((N_DEV - 1, BLK, H), jnp.bfloat16),
            pltpu.SemaphoreType.DMA((2,)),
            pltpu.SemaphoreType.DMA((N_DEV - 1,)),
            pltpu.SemaphoreType.DMA((N_DEV - 1,)),
        ],
        compiler_params=pltpu.CompilerParams(
            collective_id=0, vmem_limit_bytes=64 * 1024 * 1024),
    )(x, router_W, route_idx, expert_W, shared_W)


# device time: 89249 ns/iter; 1.8751x vs baseline; 1.2447x over previous
import jax
import jax.numpy as jnp
from jax import lax
from jax.experimental import pallas as pl
from jax.experimental.pallas import tpu as pltpu

N_DEV = 4
N_TOK = 2048
D = 1024
H = 1024
E_LOCAL = 8
E_TOT = N_DEV * E_LOCAL
BLK = N_TOK // N_DEV


def kernel(x, router_W, route_idx, expert_W, shared_W):
    def body(x_ref, router_W_ref, route_idx_ref, expert_W_ref, shared_W_ref,
             out_ref, wbuf, coef_ref, partial_ref, sbuf, rbuf, load_sems,
             send_sems, recv_sems):
        my_pos = lax.axis_index("i")

        barrier_sem = pltpu.get_barrier_semaphore()
        peers = [lax.rem(my_pos + k, N_DEV) for k in (1, 2, 3)]
        for nbr in peers:
            pl.semaphore_signal(
                barrier_sem, inc=1,
                device_id=(nbr,), device_id_type=pl.DeviceIdType.MESH)

        xf = x_ref[...]
        xh = xf.astype(jnp.bfloat16)
        xl = (xf - xh.astype(jnp.float32)).astype(jnp.bfloat16)
        rw = router_W_ref[...]
        rh = rw.astype(jnp.bfloat16)
        rl = (rw - rh.astype(jnp.float32)).astype(jnp.bfloat16)
        scores = (
            jnp.dot(xh, rh, preferred_element_type=jnp.float32)
            + jnp.dot(xh, rl, preferred_element_type=jnp.float32)
            + jnp.dot(xl, rh, preferred_element_type=jnp.float32)
        )
        m = jnp.max(scores, axis=1, keepdims=True)
        p = jnp.exp(scores - m)
        psum = jnp.sum(p, axis=1, keepdims=True)
        ridx = route_idx_ref[...]
        iota_e = lax.broadcasted_iota(jnp.int32, (N_TOK, E_TOT), 1)
        gate = jnp.sum(jnp.where(iota_e == ridx, p, 0.0), axis=1,
                       keepdims=True) / psum
        iota_l = lax.broadcasted_iota(jnp.int32, (N_TOK, E_LOCAL), 1)
        coef_ref[...] = jnp.where(
            iota_l + my_pos * E_LOCAL == ridx, gate, 0.0)

        pl.semaphore_wait(barrier_sem, N_DEV - 1)

        def wload(j, slot):
            return pltpu.make_async_copy(
                expert_W_ref.at[j], wbuf.at[slot], load_sems.at[slot])

        block_ks = (2, 1, 3, 0)
        sends = []
        own_final = None
        wload(0, 0).start()
        for j in range(E_LOCAL):
            if j + 1 < E_LOCAL:
                wload(j + 1, (j + 1) % 2).start()
            wload(j, j % 2).wait()
            for i, k in enumerate(block_ks):
                dest = lax.rem(my_pos + k, N_DEV)
                rows = pl.ds(dest * BLK, BLK)
                xm = x_ref[rows, :] * coef_ref[rows, :][:, j][:, None]
                d = jnp.dot(xm, wbuf[j % 2],
                            preferred_element_type=jnp.float32)
                if j == 0:
                    partial_ref[rows, :] = d
                elif j < E_LOCAL - 1:
                    partial_ref[rows, :] = partial_ref[rows, :] + d
                else:
                    val = partial_ref[rows, :] + d
                    if k == 0:
                        own_final = val
                    else:
                        sbuf[i] = val.astype(jnp.bfloat16)
                        slot = k - 1
                        rdma = pltpu.make_async_remote_copy(
                            src_ref=sbuf.at[i],
                            dst_ref=rbuf.at[slot],
                            send_sem=send_sems.at[i],
                            recv_sem=recv_sems.at[slot],
                            device_id=(dest,),
                            device_id_type=pl.DeviceIdType.MESH,
                        )
                        rdma.start()
                        sends.append(rdma)

        own = own_final
        own_rows = pl.ds(my_pos * BLK, BLK)
        shared_own = jnp.dot(x_ref[own_rows, :], shared_W_ref[...],
                             preferred_element_type=jnp.float32)

        for slot in range(N_DEV - 1):
            sends[slot].wait_recv()
        total = own + shared_own
        for slot in range(N_DEV - 1):
            total = total + rbuf[slot].astype(jnp.float32)
        out_ref[...] = total
        for s in sends:
            s.wait_send()

    return pl.pallas_call(
        body,
        out_shape=jax.ShapeDtypeStruct((BLK, H), jnp.float32),
        in_specs=[
            pl.BlockSpec(memory_space=pltpu.VMEM),
            pl.BlockSpec(memory_space=pltpu.VMEM),
            pl.BlockSpec(memory_space=pltpu.VMEM),
            pl.BlockSpec(memory_space=pl.ANY),
            pl.BlockSpec(memory_space=pltpu.VMEM),
        ],
        out_specs=pl.BlockSpec(memory_space=pltpu.VMEM),
        scratch_shapes=[
            pltpu.VMEM((2, D, H), jnp.float32),
            pltpu.VMEM((N_TOK, E_LOCAL), jnp.float32),
            pltpu.VMEM((N_TOK, H), jnp.float32),
            pltpu.VMEM((N_DEV - 1, BLK, H), jnp.bfloat16),
            pltpu.VMEM((N_DEV - 1, BLK, H), jnp.bfloat16),
            pltpu.SemaphoreType.DMA((2,)),
            pltpu.SemaphoreType.DMA((N_DEV - 1,)),
            pltpu.SemaphoreType.DMA((N_DEV - 1,)),
        ],
        compiler_params=pltpu.CompilerParams(
            collective_id=0, vmem_limit_bytes=64 * 1024 * 1024),
    )(x, router_W, route_idx, expert_W, shared_W)
